# baseline (speedup 1.0000x reference)
import jax
import jax.numpy as jnp
from jax import lax
from jax.experimental import pallas as pl
from jax.experimental.pallas import tpu as pltpu

_K = 8192
_D = 256
_NB = 1
_BLK = _K // _NB
_NEG = float(-3e38)
_BIG = float(3e38)


def _body(z_ref, cb_ref, cur_ref, adj_ref, out_ref, dist_ref, rows_ref, sem):
    i = pl.program_id(0)
    z = z_ref[...]
    cb = cb_ref[...]
    z2 = z.reshape(_D, 1)
    ones2 = jnp.ones((_D, 1), jnp.float32)
    a = lax.dot_general(cb, z2, (((1,), (0,)), ((), ())),
                        preferred_element_type=jnp.float32)
    b = lax.dot_general(cb * cb, ones2, (((1,), (0,)), ((), ())),
                        preferred_element_type=jnp.float32)
    dist_ref[pl.ds(i * _BLK, _BLK)] = (b - 2.0 * a).reshape(_BLK)

    @pl.when(i == _NB - 1)
    def _tail():
        d2 = dist_ref[...].reshape(64, 128)
        iota2 = lax.broadcasted_iota(jnp.int32, (64, 128), 0) * 128 + \
            lax.broadcasted_iota(jnp.int32, (64, 128), 1)

        cands = []
        cand_dists = []
        for _ in range(5):
            mn = jnp.min(d2)
            idx = jnp.min(jnp.where(d2 == mn, iota2, jnp.int32(_K)))
            cands.append(idx)
            cand_dists.append(mn)
            d2 = jnp.where(iota2 == idx, _BIG, d2)

        cur = cur_ref[0]
        copies = []
        for r in range(5):
            copies.append(pltpu.make_async_copy(
                adj_ref.at[pl.ds(cands[r], 1)], rows_ref.at[pl.ds(r, 1)],
                sem))
        copies.append(pltpu.make_async_copy(
            adj_ref.at[pl.ds(cur, 1)], rows_ref.at[pl.ds(5, 1)], sem))
        for cpy in copies:
            cpy.start()
        for cpy in copies:
            cpy.wait()

        rows = rows_ref[...]
        gdiff = jnp.mean(jnp.abs(rows[:5, :] - rows[5:6, :]), axis=1)

        best_score = jnp.full((), _NEG, jnp.float32)
        best_s = jnp.int32(0)
        for r in range(5):
            sc = -cand_dists[r] + 0.1 * gdiff[r]
            sc = jnp.where(cands[r] == cur, _NEG, sc)
            take = sc > best_score
            best_score = jnp.where(take, sc, best_score)
            best_s = jnp.where(take, cands[r], best_s)
        out_ref[0] = best_s


@jax.jit
def _run(z_flat, codebook, adjacency, cur_arr):
    out = pl.pallas_call(
        _body,
        grid=(_NB,),
        in_specs=[
            pl.BlockSpec((_D,), lambda i: (0,)),
            pl.BlockSpec((_BLK, _D), lambda i: (i, 0)),
            pl.BlockSpec(memory_space=pltpu.SMEM),
            pl.BlockSpec(memory_space=pl.ANY),
        ],
        out_specs=pl.BlockSpec(memory_space=pltpu.SMEM),
        out_shape=jax.ShapeDtypeStruct((1,), jnp.int32),
        scratch_shapes=[
            pltpu.VMEM((_K,), jnp.float32),
            pltpu.VMEM((6, _K), jnp.float32),
            pltpu.SemaphoreType.DMA,
        ],
    )(z_flat, codebook, cur_arr, adjacency)
    return out[0]


def kernel(z_flat, codebook, adjacency, current_sym):
    cur_arr = jnp.asarray(current_sym, dtype=jnp.int32).reshape(1)
    return _run(z_flat, codebook, adjacency, cur_arr)


# speedup vs baseline: 1.0674x; 1.0674x over previous
import functools

import jax
import jax.numpy as jnp
from jax import lax
from jax.experimental import pallas as pl
from jax.experimental.pallas import tpu as pltpu

_K = 8192
_D = 256
_NEG = float(-3e38)
_BIG = float(3e38)


def _body(z_ref, cb_ref, cur_ref, adj_ref, out_ref, rows_ref, sem):
    z = z_ref[...]
    cb = cb_ref[...]
    diff = cb - z[None, :]
    dists = jnp.sum(diff * diff, axis=1)
    d2 = dists.reshape(64, 128)
    iota2 = lax.broadcasted_iota(jnp.int32, (64, 128), 0) * 128 + \
        lax.broadcasted_iota(jnp.int32, (64, 128), 1)

    cands = []
    cand_dists = []
    for _ in range(5):
        m = jnp.min(d2)
        idx = jnp.min(jnp.where(d2 == m, iota2, jnp.int32(_K)))
        cands.append(idx)
        cand_dists.append(m)
        d2 = jnp.where(iota2 == idx, _BIG, d2)

    cur = cur_ref[0]
    copies = []
    for i in range(5):
        copies.append(pltpu.make_async_copy(
            adj_ref.at[pl.ds(cands[i], 1)], rows_ref.at[pl.ds(i, 1)], sem))
    copies.append(pltpu.make_async_copy(
        adj_ref.at[pl.ds(cur, 1)], rows_ref.at[pl.ds(5, 1)], sem))
    for c in copies:
        c.start()
    for c in copies:
        c.wait()

    rows = rows_ref[...]
    gdiff = jnp.mean(jnp.abs(rows[:5, :] - rows[5:6, :]), axis=1)

    best_score = jnp.full((), _NEG, jnp.float32)
    best_s = jnp.int32(0)
    for i in range(5):
        s = -cand_dists[i] + 0.1 * gdiff[i]
        s = jnp.where(cands[i] == cur, _NEG, s)
        take = s > best_score
        best_score = jnp.where(take, s, best_score)
        best_s = jnp.where(take, cands[i], best_s)
    out_ref[0] = best_s


@jax.jit
def _run(z_flat, codebook, adjacency, cur_arr):
    out = pl.pallas_call(
        _body,
        grid=(),
        in_specs=[
            pl.BlockSpec(memory_space=pltpu.VMEM),
            pl.BlockSpec(memory_space=pltpu.VMEM),
            pl.BlockSpec(memory_space=pltpu.SMEM),
            pl.BlockSpec(memory_space=pl.ANY),
        ],
        out_specs=pl.BlockSpec(memory_space=pltpu.SMEM),
        out_shape=jax.ShapeDtypeStruct((1,), jnp.int32),
        scratch_shapes=[
            pltpu.VMEM((6, _K), jnp.float32),
            pltpu.SemaphoreType.DMA,
        ],
    )(z_flat, codebook, cur_arr, adjacency)
    return out[0]


def kernel(z_flat, codebook, adjacency, current_sym):
    cur_arr = jnp.asarray(current_sym, dtype=jnp.int32).reshape(1)
    return _run(z_flat, codebook, adjacency, cur_arr)
